# baseline (speedup 1.0000x reference)
import functools

import jax
import jax.numpy as jnp
from jax import lax
from jax.experimental import pallas as pl
from jax.experimental.pallas import tpu as pltpu
from jax.experimental.pallas import tpu_sc as plsc

B = 16384
D = 200
L = 16
NC, NS = 2, 16
NW = NC * NS
RPW = B // NW
RB = L
NB = RPW // RB
NCHUNK = (D + L - 1) // L
TAIL_NEW = D - (NCHUNK - 1) * L


def _sc_body(x_hbm, out_hbm, buf, obuf):
    cid = lax.axis_index("c")
    sid = lax.axis_index("s")
    wid = sid * NC + cid
    base = wid * RPW

    pltpu.sync_copy(x_hbm.at[pl.ds(base * D, RPW * D)], buf)

    lane = lax.broadcasted_iota(jnp.int32, (L,), 0)
    ione = jnp.ones((L,), jnp.int32)
    izero = jnp.zeros((L,), jnp.int32)

    def blk(b, carry):
        r0 = b * RB
        rowbase = (r0 + lane) * D
        cnt = izero
        for j in range(D):
            v = plsc.load_gather(buf, [rowbase + j])
            cnt = cnt + jnp.where(v != 0.0, ione, izero)
        idx = rowbase + jnp.maximum(cnt - 1, 0)
        obuf[pl.ds(r0, RB)] = plsc.load_gather(buf, [idx])
        return carry

    lax.fori_loop(0, NB, blk, 0)

    pltpu.sync_copy(obuf, out_hbm.at[pl.ds(base, RPW)])


@jax.jit
def kernel(x):
    mesh = plsc.VectorSubcoreMesh(core_axis_name="c", subcore_axis_name="s")
    f = functools.partial(
        pl.kernel,
        out_type=jax.ShapeDtypeStruct((B,), jnp.float32),
        mesh=mesh,
        scratch_types=[
            pltpu.VMEM((RPW * D,), jnp.float32),
            pltpu.VMEM((RPW,), jnp.float32),
        ],
        compiler_params=pltpu.CompilerParams(needs_layout_passes=False),
    )(_sc_body)
    return f(x.reshape(-1))


# speedup vs baseline: 1.1782x; 1.1782x over previous
import functools

import jax
import jax.numpy as jnp
from jax import lax
from jax.experimental import pallas as pl
from jax.experimental.pallas import tpu as pltpu
from jax.experimental.pallas import tpu_sc as plsc

B = 16384
D = 200
L = 16
NC, NS = 2, 16
NW = NC * NS
RPW = B // NW
RB = L
NB = RPW // RB
NSPLIT = 4
CPW = RPW // NSPLIT
NCHUNK = (D + L - 1) // L
TAIL_NEW = D - (NCHUNK - 1) * L


def _sc_body(x_hbm, out_hbm, buf, obuf, s0, s1, s2, s3):
    cid = lax.axis_index("c")
    sid = lax.axis_index("s")
    wid = sid * NC + cid
    base = wid * RPW

    sems = (s0, s1, s2, s3)
    cps = [
        pltpu.async_copy(
            x_hbm.at[pl.ds((base + ci * CPW) * D, CPW * D)],
            buf.at[pl.ds(ci * CPW * D, CPW * D)],
            sems[ci],
        )
        for ci in range(NSPLIT)
    ]

    lane = lax.broadcasted_iota(jnp.int32, (L,), 0)
    tail_ok = lane >= TAIL_NEW
    izero = jnp.zeros((L,), jnp.int32)

    def blk(b, carry):
        r0 = b * RB
        cnt = izero
        for r in range(RB):
            row = r0 + r
            tot = izero
            for k in range(NCHUNK):
                off = k * L if k < NCHUNK - 1 else D - L
                v = buf[pl.ds(row * D + off, L)]
                m = (v << 1) != 0
                if k == NCHUNK - 1:
                    m = jnp.logical_and(m, tail_ok)
                tot = tot + plsc.all_reduce_population_count(m)
            cnt = jnp.where(lane == r, tot, cnt)
        idx = (r0 + lane) * D + jnp.maximum(cnt - 1, 0)
        vals = plsc.load_gather(buf, [idx])
        obuf[pl.ds(r0, RB)] = plsc.bitcast(vals, jnp.float32)
        return carry

    for ci in range(NSPLIT):
        cps[ci].wait()
        lax.fori_loop(ci * (NB // NSPLIT), (ci + 1) * (NB // NSPLIT), blk, 0)

    pltpu.sync_copy(obuf, out_hbm.at[pl.ds(base, RPW)])


@jax.jit
def kernel(x):
    mesh = plsc.VectorSubcoreMesh(core_axis_name="c", subcore_axis_name="s")
    f = functools.partial(
        pl.kernel,
        out_type=jax.ShapeDtypeStruct((B,), jnp.float32),
        mesh=mesh,
        scratch_types=[
            pltpu.VMEM((RPW * D,), jnp.int32),
            pltpu.VMEM((RPW,), jnp.float32),
            pltpu.SemaphoreType.DMA,
            pltpu.SemaphoreType.DMA,
            pltpu.SemaphoreType.DMA,
            pltpu.SemaphoreType.DMA,
        ],
        compiler_params=pltpu.CompilerParams(needs_layout_passes=False),
    )(_sc_body)
    return f(jax.lax.bitcast_convert_type(x, jnp.int32).reshape(-1))


# speedup vs baseline: 1.7016x; 1.4443x over previous
import functools

import jax
import jax.numpy as jnp
from jax import lax
from jax.experimental import pallas as pl
from jax.experimental.pallas import tpu as pltpu
from jax.experimental.pallas import tpu_sc as plsc

B = 16384
D = 200
L = 16
NC, NS = 2, 16
NW = NC * NS
RPW = B // NW
RB = L
NSPLIT = 4
CPW = RPW // NSPLIT
BPC = CPW // RB
NCHUNK = (D + L - 1) // L
TAIL_NEW = D - (NCHUNK - 1) * L


def _sc_body(x_hbm, out_hbm, buf_a, buf_b, obuf, sem_a, sem_b):
    cid = lax.axis_index("c")
    sid = lax.axis_index("s")
    wid = sid * NC + cid
    base = wid * RPW

    bufs = (buf_a, buf_b)
    sems = (sem_a, sem_b)

    lane = lax.broadcasted_iota(jnp.int32, (L,), 0)
    tail_ok = lane >= TAIL_NEW
    izero = jnp.zeros((L,), jnp.int32)

    def make_blk(buf, obuf_base):
        def blk(b, carry):
            r0 = b * RB
            cnt = izero
            for r in range(RB):
                row = r0 + r
                tot = izero
                for k in range(NCHUNK):
                    off = k * L if k < NCHUNK - 1 else D - L
                    v = buf[row, pl.ds(off, L)]
                    m = (v << 1) != 0
                    if k == NCHUNK - 1:
                        m = jnp.logical_and(m, tail_ok)
                    tot = tot + plsc.all_reduce_population_count(m)
                cnt = jnp.where(lane == r, tot, cnt)
            idx_row = r0 + lane
            idx_col = jnp.maximum(cnt - 1, 0)
            vals = plsc.load_gather(buf, [idx_row, idx_col])
            obuf[pl.ds(obuf_base + r0, RB)] = plsc.bitcast(vals, jnp.float32)
            return carry

        return blk

    cps = {}
    cps[0] = pltpu.async_copy(
        x_hbm.at[pl.ds(base, CPW)], bufs[0], sems[0]
    )
    for ci in range(NSPLIT):
        if ci + 1 < NSPLIT:
            cps[ci + 1] = pltpu.async_copy(
                x_hbm.at[pl.ds(base + (ci + 1) * CPW, CPW)],
                bufs[(ci + 1) % 2],
                sems[(ci + 1) % 2],
            )
        cps[ci].wait()
        lax.fori_loop(0, BPC, make_blk(bufs[ci % 2], ci * CPW), 0)

    pltpu.sync_copy(obuf, out_hbm.at[pl.ds(base, RPW)])


@jax.jit
def kernel(x):
    mesh = plsc.VectorSubcoreMesh(core_axis_name="c", subcore_axis_name="s")
    f = functools.partial(
        pl.kernel,
        out_type=jax.ShapeDtypeStruct((B,), jnp.float32),
        mesh=mesh,
        scratch_types=[
            pltpu.VMEM((CPW, D), jnp.int32),
            pltpu.VMEM((CPW, D), jnp.int32),
            pltpu.VMEM((RPW,), jnp.float32),
            pltpu.SemaphoreType.DMA,
            pltpu.SemaphoreType.DMA,
        ],
        compiler_params=pltpu.CompilerParams(needs_layout_passes=False),
    )(_sc_body)
    return f(jax.lax.bitcast_convert_type(x, jnp.int32))


# speedup vs baseline: 2.1334x; 1.2537x over previous
import functools

import jax
import jax.numpy as jnp
from jax import lax
from jax.experimental import pallas as pl
from jax.experimental.pallas import tpu as pltpu
from jax.experimental.pallas import tpu_sc as plsc

B = 16384
D = 200
L = 16
NC, NS = 2, 16
NW = NC * NS
RPW = B // NW
RB = L
NSPLIT = 4
CPW = RPW // NSPLIT
BPC = CPW // RB
NCHUNK = (D + L - 1) // L
TAIL_NEW = D - (NCHUNK - 1) * L


def _sc_body(x_hbm, out_hbm, buf_a, buf_b, obuf, sem_a, sem_b):
    cid = lax.axis_index("c")
    sid = lax.axis_index("s")
    wid = sid * NC + cid
    base = wid * RPW

    bufs = (buf_a, buf_b)
    sems = (sem_a, sem_b)

    lane = lax.broadcasted_iota(jnp.int32, (L,), 0)
    tail_ok = lane >= TAIL_NEW
    izero = jnp.zeros((L,), jnp.int32)

    def make_blk(buf, obuf_base):
        def blk(b, carry):
            r0 = b * RB
            cnt = izero
            for r in range(RB):
                row = r0 + r
                tot = izero
                for k in range(NCHUNK):
                    off = k * L if k < NCHUNK - 1 else D - L
                    v = plsc.bitcast(buf[row, pl.ds(off, L)], jnp.int32)
                    m = (v << 1) != 0
                    if k == NCHUNK - 1:
                        m = jnp.logical_and(m, tail_ok)
                    tot = tot + plsc.all_reduce_population_count(m)
                cnt = jnp.where(lane == r, tot, cnt)
            idx_row = r0 + lane
            idx_col = jnp.maximum(cnt - 1, 0)
            obuf[pl.ds(obuf_base + r0, RB)] = plsc.load_gather(
                buf, [idx_row, idx_col])
            return carry

        return blk

    cps = {}
    cps[0] = pltpu.async_copy(
        x_hbm.at[pl.ds(base, CPW)], bufs[0], sems[0]
    )
    for ci in range(NSPLIT):
        if ci + 1 < NSPLIT:
            cps[ci + 1] = pltpu.async_copy(
                x_hbm.at[pl.ds(base + (ci + 1) * CPW, CPW)],
                bufs[(ci + 1) % 2],
                sems[(ci + 1) % 2],
            )
        cps[ci].wait()
        lax.fori_loop(0, BPC, make_blk(bufs[ci % 2], ci * CPW), 0)

    pltpu.sync_copy(obuf, out_hbm.at[pl.ds(base, RPW)])


@jax.jit
def kernel(x):
    mesh = plsc.VectorSubcoreMesh(core_axis_name="c", subcore_axis_name="s")
    f = functools.partial(
        pl.kernel,
        out_type=jax.ShapeDtypeStruct((B,), jnp.float32),
        mesh=mesh,
        scratch_types=[
            pltpu.VMEM((CPW, D), jnp.float32),
            pltpu.VMEM((CPW, D), jnp.float32),
            pltpu.VMEM((RPW,), jnp.float32),
            pltpu.SemaphoreType.DMA,
            pltpu.SemaphoreType.DMA,
        ],
        compiler_params=pltpu.CompilerParams(needs_layout_passes=False),
    )(_sc_body)
    return f(x)


# speedup vs baseline: 2.9423x; 1.3792x over previous
import functools

import jax
import jax.numpy as jnp
from jax import lax
from jax.experimental import pallas as pl
from jax.experimental.pallas import tpu as pltpu
from jax.experimental.pallas import tpu_sc as plsc

B = 16384
D = 200
L = 16
NC, NS = 2, 16
NW = NC * NS
RPW = B // NW
NSPLIT = 2
CPW = RPW // NSPLIT
GPC = CPW // L


def _sc_body(xt_hbm, out_hbm, buf_a, buf_b, obuf, sem_a, sem_b):
    cid = lax.axis_index("c")
    sid = lax.axis_index("s")
    wid = sid * NC + cid
    base = wid * RPW

    bufs = (buf_a, buf_b)
    cps = [
        pltpu.async_copy(
            xt_hbm.at[:, pl.ds(base + ci * CPW, CPW)], bufs[ci], (sem_a, sem_b)[ci]
        )
        for ci in range(NSPLIT)
    ]

    lane = lax.broadcasted_iota(jnp.int32, (L,), 0)
    uzero = jnp.zeros((L,), jnp.uint32)
    uone = jnp.ones((L,), jnp.uint32)

    def make_grp(buf, obuf_base):
        def grp(g, carry):
            r0 = g * L
            acc = [uzero, uzero, uzero, uzero]
            for c in range(D):
                v = plsc.bitcast(buf[c, pl.ds(r0, L)], jnp.uint32)
                acc[c % 4] = acc[c % 4] + jnp.minimum(v << 1, uone)
            cnt = plsc.bitcast((acc[0] + acc[1]) + (acc[2] + acc[3]), jnp.int32)
            idx_c = jnp.maximum(cnt - 1, 0)
            obuf[pl.ds(obuf_base + r0, L)] = plsc.load_gather(
                buf, [idx_c, r0 + lane]
            )
            return carry

        return grp

    for ci in range(NSPLIT):
        cps[ci].wait()
        lax.fori_loop(0, GPC, make_grp(bufs[ci], ci * CPW), 0)

    pltpu.sync_copy(obuf, out_hbm.at[pl.ds(base, RPW)])


@jax.jit
def kernel(x):
    mesh = plsc.VectorSubcoreMesh(core_axis_name="c", subcore_axis_name="s")
    f = functools.partial(
        pl.kernel,
        out_type=jax.ShapeDtypeStruct((B,), jnp.float32),
        mesh=mesh,
        scratch_types=[
            pltpu.VMEM((D, CPW), jnp.float32),
            pltpu.VMEM((D, CPW), jnp.float32),
            pltpu.VMEM((RPW,), jnp.float32),
            pltpu.SemaphoreType.DMA,
            pltpu.SemaphoreType.DMA,
        ],
        compiler_params=pltpu.CompilerParams(needs_layout_passes=False),
    )(_sc_body)
    return f(x.T)
